# baseline (speedup 1.0000x reference)
import jax
import jax.numpy as jnp
from jax.experimental import pallas as pl
from jax.experimental.pallas import tpu as pltpu

B, L, D = 4, 2048, 1024
KSIZE = 25
PAD = KSIZE // 2
TOPK = 16
EPS = 1e-5
CBLK = 128


def _topk_idx_kernel(corr_ref, out_ref, mag_ref, idxs_ref):
    mag_ref[...] = jnp.abs(corr_ref[0])
    rows = jax.lax.broadcasted_iota(jnp.int32, (L, CBLK), 0)

    def body(k, carry):
        mag = mag_ref[...]
        mx = jnp.max(mag, axis=0)
        idx = jnp.min(jnp.where(mag == mx[None, :], rows, L), axis=0)
        mag_ref[...] = jnp.where(rows == idx[None, :], -1.0, mag)
        idxs_ref[pl.ds(k, 1), :] = idx[None, :]
        return carry

    jax.lax.fori_loop(0, TOPK, body, 0)
    out_ref[0] = idxs_ref[...].T


def _topk_idx(corr):
    return pl.pallas_call(
        _topk_idx_kernel,
        grid=(B, D // CBLK),
        in_specs=[pl.BlockSpec((1, L, CBLK), lambda b, c: (b, 0, c))],
        out_specs=pl.BlockSpec((1, CBLK, TOPK), lambda b, c: (b, c, 0)),
        out_shape=jax.ShapeDtypeStruct((B, D, TOPK), jnp.int32),
        scratch_shapes=[pltpu.VMEM((L, CBLK), jnp.float32),
                        pltpu.VMEM((TOPK, CBLK), jnp.int32)],
    )(corr)


def _topk_a_kernel(corr_ref, a_ref, mag_ref):
    mag_ref[...] = jnp.abs(corr_ref[0])
    rows = jax.lax.broadcasted_iota(jnp.int32, (L, CBLK), 0)

    def body(_, carry):
        mag = mag_ref[...]
        mx = jnp.max(mag, axis=0)
        idx = jnp.min(jnp.where(mag == mx[None, :], rows, L), axis=0)
        mag_ref[...] = jnp.where(rows == idx[None, :], -1.0, mag)
        return carry

    jax.lax.fori_loop(0, TOPK, body, 0)
    a_ref[0] = jnp.where(mag_ref[...] < 0, corr_ref[0], 0.0)


def _topk_a(corr):
    return pl.pallas_call(
        _topk_a_kernel,
        grid=(B, D // CBLK),
        in_specs=[pl.BlockSpec((1, L, CBLK), lambda b, c: (b, 0, c))],
        out_specs=pl.BlockSpec((1, L, CBLK), lambda b, c: (b, 0, c)),
        out_shape=jax.ShapeDtypeStruct((B, L, D), jnp.float32),
        scratch_shapes=[pltpu.VMEM((L, CBLK), jnp.float32)],
    )(corr)


RB = 512
NC = D // CBLK


def _lin_kernel(a_ref, w_ref, o_ref):
    c = pl.program_id(1)
    partial = jax.lax.dot_general(
        a_ref[0], w_ref[...], (((1,), (0,)), ((), ())),
        precision=jax.lax.Precision.HIGHEST,
        preferred_element_type=jnp.float32)

    @pl.when(c == 0)
    def _():
        o_ref[0] = partial

    @pl.when(c != 0)
    def _():
        o_ref[0] = o_ref[0] + partial


def _lin(a, W1T):
    return pl.pallas_call(
        _lin_kernel,
        grid=(B, NC),
        in_specs=[pl.BlockSpec((1, L, CBLK), lambda b, c: (b, 0, c)),
                  pl.BlockSpec((CBLK, D), lambda b, c: (c, 0))],
        out_specs=pl.BlockSpec((1, L, D), lambda b, c: (b, 0, 0)),
        out_shape=jax.ShapeDtypeStruct((B, L, D), jnp.float32),
    )(a, W1T)


def _ln2_kernel(o_ref, s1_ref, trend_ref, b1_ref, g1_ref, be1_ref,
                gF_ref, beF_ref, out_ref):
    y = s1_ref[0] + (o_ref[0] + b1_ref[...])
    mu = jnp.mean(y, axis=-1, keepdims=True)
    var = jnp.mean((y - mu) ** 2, axis=-1, keepdims=True)
    s2 = (y - mu) / jnp.sqrt(var + EPS) * g1_ref[...] + be1_ref[...]
    z = s2 + trend_ref[0]
    mu2 = jnp.mean(z, axis=-1, keepdims=True)
    var2 = jnp.mean((z - mu2) ** 2, axis=-1, keepdims=True)
    out_ref[0] = (z - mu2) / jnp.sqrt(var2 + EPS) * gF_ref[...] + beF_ref[...]


def _ln2(o, s1, trend, b1, g1, be1, gF, beF):
    vec = lambda v: jnp.reshape(v, (1, D))
    return pl.pallas_call(
        _ln2_kernel,
        grid=(B, L // RB),
        in_specs=[pl.BlockSpec((1, RB, D), lambda b, r: (b, r, 0))] * 3
        + [pl.BlockSpec((1, D), lambda b, r: (0, 0))] * 5,
        out_specs=pl.BlockSpec((1, RB, D), lambda b, r: (b, r, 0)),
        out_shape=jax.ShapeDtypeStruct((B, L, D), jnp.float32),
    )(o, s1, trend, vec(b1), vec(g1), vec(be1), vec(gF), vec(beF))


def _decomp_kernel(x_ref, trend_ref, seas_ref, xp_ref):
    CB = x_ref.shape[2]
    xp_ref[PAD:PAD + L, :] = x_ref[0]
    for j in range(1, PAD + 1):
        xp_ref[PAD - j, :] = x_ref[0, j, :]
        xp_ref[PAD + L - 1 + j, :] = x_ref[0, L - 1 - j, :]
    RC = 256
    for r in range(0, L, RC):
        acc = xp_ref[r:r + RC, :]
        for i in range(1, KSIZE):
            acc = acc + xp_ref[r + i:r + i + RC, :]
        trend = acc / KSIZE
        trend_ref[0, r:r + RC, :] = trend
        seas_ref[0, r:r + RC, :] = x_ref[0, r:r + RC, :] - trend


def _decomp(x):
    CB = 128
    return pl.pallas_call(
        _decomp_kernel,
        grid=(B, D // CB),
        in_specs=[pl.BlockSpec((1, L, CB), lambda b, c: (b, 0, c))],
        out_specs=[pl.BlockSpec((1, L, CB), lambda b, c: (b, 0, c)),
                   pl.BlockSpec((1, L, CB), lambda b, c: (b, 0, c))],
        out_shape=[jax.ShapeDtypeStruct((B, L, D), jnp.float32),
                   jax.ShapeDtypeStruct((B, L, D), jnp.float32)],
        scratch_shapes=[pltpu.VMEM((L + 2 * PAD + 8, CB), jnp.float32)],
    )(x)


def _layer_norm(x, g, b):
    mu = jnp.mean(x, axis=-1, keepdims=True)
    var = jnp.mean((x - mu) ** 2, axis=-1, keepdims=True)
    return (x - mu) / jnp.sqrt(var + EPS) * g + b


def kernel(x, W0, b0, g0, be0, W1, b1, g1, be1, gF, beF):
    trend, seasonal = _decomp(x)

    bidx = jnp.arange(B)[:, None, None]
    cidx = jnp.arange(D)[None, :, None]
    Xf = jnp.fft.rfft(seasonal, axis=1)
    ACf = Xf * jnp.conj(Xf)
    corr = jnp.fft.irfft(ACf, n=L, axis=1)
    corr_t = jnp.transpose(corr, (0, 2, 1))
    idx = _topk_idx(corr)
    mask = jnp.zeros(corr_t.shape, corr_t.dtype).at[bidx, cidx, idx].set(1.0)
    a = jnp.transpose(corr_t * mask, (0, 2, 1))
    o = a @ W0.T + b0
    s1 = _layer_norm(seasonal + o, g0, be0)

    Xf2 = jnp.fft.rfft(s1, axis=1)
    ACf2 = Xf2 * jnp.conj(Xf2)
    corr2 = jnp.fft.irfft(ACf2, n=L, axis=1)
    a2 = _topk_a(corr2)
    o2 = _lin(a2, jnp.transpose(W1))
    return _ln2(o2, s1, trend, b1, g1, be1, gF, beF)


# speedup vs baseline: 1.1097x; 1.1097x over previous
import jax
import jax.numpy as jnp
from jax.experimental import pallas as pl
from jax.experimental.pallas import tpu as pltpu

B, L, D = 4, 2048, 1024
KSIZE = 25
PAD = KSIZE // 2
TOPK = 16
EPS = 1e-5
CBLK = 128
CTOP = 512


def _topk_idx_kernel(corr_ref, out_ref, mag_ref, idxs_ref):
    CB = corr_ref.shape[2]
    mag_ref[...] = jnp.abs(corr_ref[0])
    rows = jax.lax.broadcasted_iota(jnp.int32, (L, CB), 0)

    def body(k, carry):
        mag = mag_ref[...]
        mx = jnp.max(mag, axis=0)
        idx = jnp.min(jnp.where(mag == mx[None, :], rows, L), axis=0)
        mag_ref[...] = jnp.where(rows == idx[None, :], -1.0, mag)
        idxs_ref[pl.ds(k, 1), :] = idx[None, :]
        return carry

    jax.lax.fori_loop(0, TOPK, body, 0)
    out_ref[0] = idxs_ref[...].T


def _topk_idx(corr):
    return pl.pallas_call(
        _topk_idx_kernel,
        grid=(B, D // CTOP),
        in_specs=[pl.BlockSpec((1, L, CTOP), lambda b, c: (b, 0, c))],
        out_specs=pl.BlockSpec((1, CTOP, TOPK), lambda b, c: (b, c, 0)),
        out_shape=jax.ShapeDtypeStruct((B, D, TOPK), jnp.int32),
        scratch_shapes=[pltpu.VMEM((L, CTOP), jnp.float32),
                        pltpu.VMEM((TOPK, CTOP), jnp.int32)],
    )(corr)


def _topk_a_kernel(corr_ref, a_ref, mag_ref):
    CB = corr_ref.shape[2]
    mag_ref[...] = jnp.abs(corr_ref[0])
    rows = jax.lax.broadcasted_iota(jnp.int32, (L, CB), 0)

    def body(_, carry):
        mag = mag_ref[...]
        mx = jnp.max(mag, axis=0)
        idx = jnp.min(jnp.where(mag == mx[None, :], rows, L), axis=0)
        mag_ref[...] = jnp.where(rows == idx[None, :], -1.0, mag)
        return carry

    jax.lax.fori_loop(0, TOPK, body, 0)
    a_ref[0] = jnp.where(mag_ref[...] < 0, corr_ref[0], 0.0)


def _topk_a(corr):
    return pl.pallas_call(
        _topk_a_kernel,
        grid=(B, D // CTOP),
        in_specs=[pl.BlockSpec((1, L, CTOP), lambda b, c: (b, 0, c))],
        out_specs=pl.BlockSpec((1, L, CTOP), lambda b, c: (b, 0, c)),
        out_shape=jax.ShapeDtypeStruct((B, L, D), jnp.float32),
        scratch_shapes=[pltpu.VMEM((L, CTOP), jnp.float32)],
    )(corr)


RB = 512
NC = D // CBLK


def _lin_kernel(a_ref, w_ref, o_ref):
    c = pl.program_id(1)
    partial = jax.lax.dot_general(
        a_ref[0], w_ref[...], (((1,), (0,)), ((), ())),
        precision=jax.lax.Precision.DEFAULT,
        preferred_element_type=jnp.float32)

    @pl.when(c == 0)
    def _():
        o_ref[0] = partial

    @pl.when(c != 0)
    def _():
        o_ref[0] = o_ref[0] + partial


def _lin(a, W1T):
    return pl.pallas_call(
        _lin_kernel,
        grid=(B, NC),
        in_specs=[pl.BlockSpec((1, L, CBLK), lambda b, c: (b, 0, c)),
                  pl.BlockSpec((CBLK, D), lambda b, c: (c, 0))],
        out_specs=pl.BlockSpec((1, L, D), lambda b, c: (b, 0, 0)),
        out_shape=jax.ShapeDtypeStruct((B, L, D), jnp.float32),
    )(a, W1T)


def _ln2_kernel(o_ref, s1_ref, trend_ref, b1_ref, g1_ref, be1_ref,
                gF_ref, beF_ref, out_ref):
    y = s1_ref[0] + (o_ref[0] + b1_ref[...])
    mu = jnp.mean(y, axis=-1, keepdims=True)
    var = jnp.mean((y - mu) ** 2, axis=-1, keepdims=True)
    s2 = (y - mu) / jnp.sqrt(var + EPS) * g1_ref[...] + be1_ref[...]
    z = s2 + trend_ref[0]
    mu2 = jnp.mean(z, axis=-1, keepdims=True)
    var2 = jnp.mean((z - mu2) ** 2, axis=-1, keepdims=True)
    out_ref[0] = (z - mu2) / jnp.sqrt(var2 + EPS) * gF_ref[...] + beF_ref[...]


def _ln2(o, s1, trend, b1, g1, be1, gF, beF):
    vec = lambda v: jnp.reshape(v, (1, D))
    return pl.pallas_call(
        _ln2_kernel,
        grid=(B, L // RB),
        in_specs=[pl.BlockSpec((1, RB, D), lambda b, r: (b, r, 0))] * 3
        + [pl.BlockSpec((1, D), lambda b, r: (0, 0))] * 5,
        out_specs=pl.BlockSpec((1, RB, D), lambda b, r: (b, r, 0)),
        out_shape=jax.ShapeDtypeStruct((B, L, D), jnp.float32),
    )(o, s1, trend, vec(b1), vec(g1), vec(be1), vec(gF), vec(beF))


def _decomp_kernel(x_ref, trend_ref, seas_ref, xp_ref):
    CB = x_ref.shape[2]
    xp_ref[PAD:PAD + L, :] = x_ref[0]
    for j in range(1, PAD + 1):
        xp_ref[PAD - j, :] = x_ref[0, j, :]
        xp_ref[PAD + L - 1 + j, :] = x_ref[0, L - 1 - j, :]
    RC = 256
    for r in range(0, L, RC):
        acc = xp_ref[r:r + RC, :]
        for i in range(1, KSIZE):
            acc = acc + xp_ref[r + i:r + i + RC, :]
        trend = acc / KSIZE
        trend_ref[0, r:r + RC, :] = trend
        seas_ref[0, r:r + RC, :] = x_ref[0, r:r + RC, :] - trend


def _decomp(x):
    CB = 128
    return pl.pallas_call(
        _decomp_kernel,
        grid=(B, D // CB),
        in_specs=[pl.BlockSpec((1, L, CB), lambda b, c: (b, 0, c))],
        out_specs=[pl.BlockSpec((1, L, CB), lambda b, c: (b, 0, c)),
                   pl.BlockSpec((1, L, CB), lambda b, c: (b, 0, c))],
        out_shape=[jax.ShapeDtypeStruct((B, L, D), jnp.float32),
                   jax.ShapeDtypeStruct((B, L, D), jnp.float32)],
        scratch_shapes=[pltpu.VMEM((L + 2 * PAD + 8, CB), jnp.float32)],
    )(x)


def _layer_norm(x, g, b):
    mu = jnp.mean(x, axis=-1, keepdims=True)
    var = jnp.mean((x - mu) ** 2, axis=-1, keepdims=True)
    return (x - mu) / jnp.sqrt(var + EPS) * g + b


def kernel(x, W0, b0, g0, be0, W1, b1, g1, be1, gF, beF):
    trend, seasonal = _decomp(x)

    bidx = jnp.arange(B)[:, None, None]
    cidx = jnp.arange(D)[None, :, None]
    Xf = jnp.fft.rfft(seasonal, axis=1)
    ACf = Xf * jnp.conj(Xf)
    corr = jnp.fft.irfft(ACf, n=L, axis=1)
    corr_t = jnp.transpose(corr, (0, 2, 1))
    idx = _topk_idx(corr)
    mask = jnp.zeros(corr_t.shape, corr_t.dtype).at[bidx, cidx, idx].set(1.0)
    a = jnp.transpose(corr_t * mask, (0, 2, 1))
    o = a @ W0.T + b0
    s1 = _layer_norm(seasonal + o, g0, be0)

    Xf2 = jnp.fft.rfft(s1, axis=1)
    ACf2 = Xf2 * jnp.conj(Xf2)
    corr2 = jnp.fft.irfft(ACf2, n=L, axis=1)
    a2 = _topk_a(corr2)
    o2 = _lin(a2, jnp.transpose(W1))
    return _ln2(o2, s1, trend, b1, g1, be1, gF, beF)


# speedup vs baseline: 1.1269x; 1.0155x over previous
import jax
import jax.numpy as jnp
from jax.experimental import pallas as pl
from jax.experimental.pallas import tpu as pltpu

B, L, D = 4, 2048, 1024
KSIZE = 25
PAD = KSIZE // 2
TOPK = 16
EPS = 1e-5
CBLK = 128
CTOP = 512


def _topk_idx_kernel(corr_ref, out_ref, mag_ref, idxs_ref):
    CB = corr_ref.shape[2]
    mag_ref[...] = jnp.abs(corr_ref[0])
    rows = jax.lax.broadcasted_iota(jnp.int32, (L, CB), 0)

    def body(k, carry):
        mag = mag_ref[...]
        mx = jnp.max(mag, axis=0)
        idx = jnp.min(jnp.where(mag == mx[None, :], rows, L), axis=0)
        mag_ref[...] = jnp.where(rows == idx[None, :], -1.0, mag)
        idxs_ref[pl.ds(k, 1), :] = idx[None, :]
        return carry

    jax.lax.fori_loop(0, TOPK, body, 0)
    out_ref[0] = idxs_ref[...].T


def _topk_idx(corr):
    return pl.pallas_call(
        _topk_idx_kernel,
        grid=(B, D // CTOP),
        in_specs=[pl.BlockSpec((1, L, CTOP), lambda b, c: (b, 0, c))],
        out_specs=pl.BlockSpec((1, CTOP, TOPK), lambda b, c: (b, c, 0)),
        out_shape=jax.ShapeDtypeStruct((B, D, TOPK), jnp.int32),
        scratch_shapes=[pltpu.VMEM((L, CTOP), jnp.float32),
                        pltpu.VMEM((TOPK, CTOP), jnp.int32)],
    )(corr)


RB = 512


def _topk_lin_kernel(corr_ref, w_ref, o_ref, mag_ref):
    c = pl.program_id(1)
    CB = corr_ref.shape[2]
    mag_ref[...] = jnp.abs(corr_ref[0])
    rows = jax.lax.broadcasted_iota(jnp.int32, (L, CB), 0)

    def body(_, carry):
        mag = mag_ref[...]
        mx = jnp.max(mag, axis=0)
        idx = jnp.min(jnp.where(mag == mx[None, :], rows, L), axis=0)
        mag_ref[...] = jnp.where(rows == idx[None, :], -1.0, mag)
        return carry

    jax.lax.fori_loop(0, TOPK, body, 0)
    a = jnp.where(mag_ref[...] < 0, corr_ref[0], 0.0)
    partial = jax.lax.dot_general(
        a, w_ref[...], (((1,), (0,)), ((), ())),
        precision=jax.lax.Precision.DEFAULT,
        preferred_element_type=jnp.float32)

    @pl.when(c == 0)
    def _():
        o_ref[0] = partial

    @pl.when(c != 0)
    def _():
        o_ref[0] = o_ref[0] + partial


def _topk_lin(corr, W1T):
    return pl.pallas_call(
        _topk_lin_kernel,
        grid=(B, D // CTOP),
        in_specs=[pl.BlockSpec((1, L, CTOP), lambda b, c: (b, 0, c)),
                  pl.BlockSpec((CTOP, D), lambda b, c: (c, 0))],
        out_specs=pl.BlockSpec((1, L, D), lambda b, c: (b, 0, 0)),
        out_shape=jax.ShapeDtypeStruct((B, L, D), jnp.float32),
        scratch_shapes=[pltpu.VMEM((L, CTOP), jnp.float32)],
    )(corr, W1T)


def _ln2_kernel(o_ref, s1_ref, trend_ref, b1_ref, g1_ref, be1_ref,
                gF_ref, beF_ref, out_ref):
    y = s1_ref[0] + (o_ref[0] + b1_ref[...])
    mu = jnp.mean(y, axis=-1, keepdims=True)
    var = jnp.mean((y - mu) ** 2, axis=-1, keepdims=True)
    s2 = (y - mu) / jnp.sqrt(var + EPS) * g1_ref[...] + be1_ref[...]
    z = s2 + trend_ref[0]
    mu2 = jnp.mean(z, axis=-1, keepdims=True)
    var2 = jnp.mean((z - mu2) ** 2, axis=-1, keepdims=True)
    out_ref[0] = (z - mu2) / jnp.sqrt(var2 + EPS) * gF_ref[...] + beF_ref[...]


def _ln2(o, s1, trend, b1, g1, be1, gF, beF):
    vec = lambda v: jnp.reshape(v, (1, D))
    return pl.pallas_call(
        _ln2_kernel,
        grid=(B, L // RB),
        in_specs=[pl.BlockSpec((1, RB, D), lambda b, r: (b, r, 0))] * 3
        + [pl.BlockSpec((1, D), lambda b, r: (0, 0))] * 5,
        out_specs=pl.BlockSpec((1, RB, D), lambda b, r: (b, r, 0)),
        out_shape=jax.ShapeDtypeStruct((B, L, D), jnp.float32),
    )(o, s1, trend, vec(b1), vec(g1), vec(be1), vec(gF), vec(beF))


def _decomp_kernel(x_ref, trend_ref, seas_ref, xp_ref):
    CB = x_ref.shape[2]
    xp_ref[PAD:PAD + L, :] = x_ref[0]
    for j in range(1, PAD + 1):
        xp_ref[PAD - j, :] = x_ref[0, j, :]
        xp_ref[PAD + L - 1 + j, :] = x_ref[0, L - 1 - j, :]
    RC = 256
    for r in range(0, L, RC):
        acc = xp_ref[r:r + RC, :]
        for i in range(1, KSIZE):
            acc = acc + xp_ref[r + i:r + i + RC, :]
        trend = acc / KSIZE
        trend_ref[0, r:r + RC, :] = trend
        seas_ref[0, r:r + RC, :] = x_ref[0, r:r + RC, :] - trend


def _decomp(x):
    CB = 128
    return pl.pallas_call(
        _decomp_kernel,
        grid=(B, D // CB),
        in_specs=[pl.BlockSpec((1, L, CB), lambda b, c: (b, 0, c))],
        out_specs=[pl.BlockSpec((1, L, CB), lambda b, c: (b, 0, c)),
                   pl.BlockSpec((1, L, CB), lambda b, c: (b, 0, c))],
        out_shape=[jax.ShapeDtypeStruct((B, L, D), jnp.float32),
                   jax.ShapeDtypeStruct((B, L, D), jnp.float32)],
        scratch_shapes=[pltpu.VMEM((L + 2 * PAD + 8, CB), jnp.float32)],
    )(x)


def _layer_norm(x, g, b):
    mu = jnp.mean(x, axis=-1, keepdims=True)
    var = jnp.mean((x - mu) ** 2, axis=-1, keepdims=True)
    return (x - mu) / jnp.sqrt(var + EPS) * g + b


def kernel(x, W0, b0, g0, be0, W1, b1, g1, be1, gF, beF):
    trend, seasonal = _decomp(x)

    bidx = jnp.arange(B)[:, None, None]
    cidx = jnp.arange(D)[None, :, None]
    Xf = jnp.fft.rfft(seasonal, axis=1)
    ACf = Xf * jnp.conj(Xf)
    corr = jnp.fft.irfft(ACf, n=L, axis=1)
    corr_t = jnp.transpose(corr, (0, 2, 1))
    idx = _topk_idx(corr)
    mask = jnp.zeros(corr_t.shape, corr_t.dtype).at[bidx, cidx, idx].set(1.0)
    a = jnp.transpose(corr_t * mask, (0, 2, 1))
    o = a @ W0.T + b0
    s1 = _layer_norm(seasonal + o, g0, be0)

    Xf2 = jnp.fft.rfft(s1, axis=1)
    ACf2 = Xf2 * jnp.conj(Xf2)
    corr2 = jnp.fft.irfft(ACf2, n=L, axis=1)
    o2 = _topk_lin(corr2, jnp.transpose(W1))
    return _ln2(o2, s1, trend, b1, g1, be1, gF, beF)


# speedup vs baseline: 1.1326x; 1.0051x over previous
import jax
import jax.numpy as jnp
from jax.experimental import pallas as pl
from jax.experimental.pallas import tpu as pltpu

B, L, D = 4, 2048, 1024
KSIZE = 25
PAD = KSIZE // 2
TOPK = 16
EPS = 1e-5
CBLK = 128
CTOP = 512


def _topk_idx_kernel(corr_ref, out_ref, mag_ref, idxs_ref):
    CB = corr_ref.shape[2]
    mag_ref[...] = jnp.abs(corr_ref[0])
    rows = jax.lax.broadcasted_iota(jnp.int32, (L, CB), 0)

    def body(k, carry):
        mag = mag_ref[...]
        mx = jnp.max(mag, axis=0)
        idx = jnp.min(jnp.where(mag == mx[None, :], rows, L), axis=0)
        mag_ref[...] = jnp.where(rows == idx[None, :], -1.0, mag)
        idxs_ref[pl.ds(k, 1), :] = idx[None, :]
        return carry

    jax.lax.fori_loop(0, TOPK, body, 0)
    out_ref[0] = idxs_ref[...].T


def _topk_idx(corr):
    return pl.pallas_call(
        _topk_idx_kernel,
        grid=(B, 1),
        in_specs=[pl.BlockSpec((1, L, D), lambda b, c: (b, 0, 0))],
        out_specs=pl.BlockSpec((1, D, TOPK), lambda b, c: (b, 0, 0)),
        out_shape=jax.ShapeDtypeStruct((B, D, TOPK), jnp.int32),
        scratch_shapes=[pltpu.VMEM((L, D), jnp.float32),
                        pltpu.VMEM((TOPK, D), jnp.int32)],
    )(corr)


RB = 1024


def _topk_lin_kernel(corr_ref, w_ref, o_ref, mag_ref):
    c = pl.program_id(1)
    CB = corr_ref.shape[2]
    mag_ref[...] = jnp.abs(corr_ref[0])
    rows = jax.lax.broadcasted_iota(jnp.int32, (L, CB), 0)

    def body(_, carry):
        mag = mag_ref[...]
        mx = jnp.max(mag, axis=0)
        idx = jnp.min(jnp.where(mag == mx[None, :], rows, L), axis=0)
        mag_ref[...] = jnp.where(rows == idx[None, :], -1.0, mag)
        return carry

    jax.lax.fori_loop(0, TOPK, body, 0)
    a = jnp.where(mag_ref[...] < 0, corr_ref[0], 0.0)
    partial = jax.lax.dot_general(
        a, w_ref[...], (((1,), (0,)), ((), ())),
        precision=jax.lax.Precision.DEFAULT,
        preferred_element_type=jnp.float32)

    @pl.when(c == 0)
    def _():
        o_ref[0] = partial

    @pl.when(c != 0)
    def _():
        o_ref[0] = o_ref[0] + partial


def _topk_lin(corr, W1T):
    return pl.pallas_call(
        _topk_lin_kernel,
        grid=(B, D // CTOP),
        in_specs=[pl.BlockSpec((1, L, CTOP), lambda b, c: (b, 0, c)),
                  pl.BlockSpec((CTOP, D), lambda b, c: (c, 0))],
        out_specs=pl.BlockSpec((1, L, D), lambda b, c: (b, 0, 0)),
        out_shape=jax.ShapeDtypeStruct((B, L, D), jnp.float32),
        scratch_shapes=[pltpu.VMEM((L, CTOP), jnp.float32)],
    )(corr, W1T)


def _ln2_kernel(o_ref, s1_ref, trend_ref, b1_ref, g1_ref, be1_ref,
                gF_ref, beF_ref, out_ref):
    y = s1_ref[0] + (o_ref[0] + b1_ref[...])
    mu = jnp.mean(y, axis=-1, keepdims=True)
    var = jnp.mean((y - mu) ** 2, axis=-1, keepdims=True)
    s2 = (y - mu) / jnp.sqrt(var + EPS) * g1_ref[...] + be1_ref[...]
    z = s2 + trend_ref[0]
    mu2 = jnp.mean(z, axis=-1, keepdims=True)
    var2 = jnp.mean((z - mu2) ** 2, axis=-1, keepdims=True)
    out_ref[0] = (z - mu2) / jnp.sqrt(var2 + EPS) * gF_ref[...] + beF_ref[...]


def _ln2(o, s1, trend, b1, g1, be1, gF, beF):
    vec = lambda v: jnp.reshape(v, (1, D))
    return pl.pallas_call(
        _ln2_kernel,
        grid=(B, L // RB),
        in_specs=[pl.BlockSpec((1, RB, D), lambda b, r: (b, r, 0))] * 3
        + [pl.BlockSpec((1, D), lambda b, r: (0, 0))] * 5,
        out_specs=pl.BlockSpec((1, RB, D), lambda b, r: (b, r, 0)),
        out_shape=jax.ShapeDtypeStruct((B, L, D), jnp.float32),
    )(o, s1, trend, vec(b1), vec(g1), vec(be1), vec(gF), vec(beF))


def _decomp_kernel(x_ref, trend_ref, seas_ref, xp_ref):
    CB = x_ref.shape[2]
    xp_ref[PAD:PAD + L, :] = x_ref[0]
    for j in range(1, PAD + 1):
        xp_ref[PAD - j, :] = x_ref[0, j, :]
        xp_ref[PAD + L - 1 + j, :] = x_ref[0, L - 1 - j, :]
    RC = 256
    for r in range(0, L, RC):
        acc = xp_ref[r:r + RC, :]
        for i in range(1, KSIZE):
            acc = acc + xp_ref[r + i:r + i + RC, :]
        trend = acc / KSIZE
        trend_ref[0, r:r + RC, :] = trend
        seas_ref[0, r:r + RC, :] = x_ref[0, r:r + RC, :] - trend


def _decomp(x):
    CB = 512
    return pl.pallas_call(
        _decomp_kernel,
        grid=(B, D // CB),
        in_specs=[pl.BlockSpec((1, L, CB), lambda b, c: (b, 0, c))],
        out_specs=[pl.BlockSpec((1, L, CB), lambda b, c: (b, 0, c)),
                   pl.BlockSpec((1, L, CB), lambda b, c: (b, 0, c))],
        out_shape=[jax.ShapeDtypeStruct((B, L, D), jnp.float32),
                   jax.ShapeDtypeStruct((B, L, D), jnp.float32)],
        scratch_shapes=[pltpu.VMEM((L + 2 * PAD + 8, CB), jnp.float32)],
    )(x)


def _layer_norm(x, g, b):
    mu = jnp.mean(x, axis=-1, keepdims=True)
    var = jnp.mean((x - mu) ** 2, axis=-1, keepdims=True)
    return (x - mu) / jnp.sqrt(var + EPS) * g + b


def kernel(x, W0, b0, g0, be0, W1, b1, g1, be1, gF, beF):
    trend, seasonal = _decomp(x)

    bidx = jnp.arange(B)[:, None, None]
    cidx = jnp.arange(D)[None, :, None]
    Xf = jnp.fft.rfft(seasonal, axis=1)
    ACf = Xf * jnp.conj(Xf)
    corr = jnp.fft.irfft(ACf, n=L, axis=1)
    corr_t = jnp.transpose(corr, (0, 2, 1))
    idx = _topk_idx(corr)
    mask = jnp.zeros(corr_t.shape, corr_t.dtype).at[bidx, cidx, idx].set(1.0)
    a = jnp.transpose(corr_t * mask, (0, 2, 1))
    o = a @ W0.T + b0
    s1 = _layer_norm(seasonal + o, g0, be0)

    Xf2 = jnp.fft.rfft(s1, axis=1)
    ACf2 = Xf2 * jnp.conj(Xf2)
    corr2 = jnp.fft.irfft(ACf2, n=L, axis=1)
    o2 = _topk_lin(corr2, jnp.transpose(W1))
    return _ln2(o2, s1, trend, b1, g1, be1, gF, beF)


# speedup vs baseline: 1.1563x; 1.0209x over previous
import jax
import jax.numpy as jnp
from jax.experimental import pallas as pl
from jax.experimental.pallas import tpu as pltpu

B, L, D = 4, 2048, 1024
KSIZE = 25
PAD = KSIZE // 2
TOPK = 16
EPS = 1e-5
CBLK = 128
CTOP = 512


def _topk_idx_kernel(corr_ref, out_ref, mag_ref, idxs_ref):
    CB = corr_ref.shape[2]
    mag_ref[...] = jnp.abs(corr_ref[0])
    rows = jax.lax.broadcasted_iota(jnp.int32, (L, CB), 0)

    def body(k, mx):
        mag = mag_ref[...]
        idx = jnp.min(jnp.where(mag == mx[None, :], rows, L), axis=0)
        idxs_ref[pl.ds(k, 1), :] = idx[None, :]
        mag2 = jnp.where(rows == idx[None, :], -1.0, mag)
        mag_ref[...] = mag2
        return jnp.max(mag2, axis=0)

    jax.lax.fori_loop(0, TOPK, body, jnp.max(mag_ref[...], axis=0))
    out_ref[0] = idxs_ref[...].T


def _topk_idx(corr):
    return pl.pallas_call(
        _topk_idx_kernel,
        grid=(B, 1),
        in_specs=[pl.BlockSpec((1, L, D), lambda b, c: (b, 0, 0))],
        out_specs=pl.BlockSpec((1, D, TOPK), lambda b, c: (b, 0, 0)),
        out_shape=jax.ShapeDtypeStruct((B, D, TOPK), jnp.int32),
        scratch_shapes=[pltpu.VMEM((L, D), jnp.float32),
                        pltpu.VMEM((TOPK, D), jnp.int32)],
    )(corr)


RB = 1024


def _topk_lin_kernel(corr_ref, w_ref, o_ref, mag_ref):
    c = pl.program_id(1)
    CB = corr_ref.shape[2]
    mag_ref[...] = jnp.abs(corr_ref[0])
    rows = jax.lax.broadcasted_iota(jnp.int32, (L, CB), 0)

    def body(_, mx):
        mag = mag_ref[...]
        idx = jnp.min(jnp.where(mag == mx[None, :], rows, L), axis=0)
        mag2 = jnp.where(rows == idx[None, :], -1.0, mag)
        mag_ref[...] = mag2
        return jnp.max(mag2, axis=0)

    jax.lax.fori_loop(0, TOPK, body, jnp.max(mag_ref[...], axis=0))
    a = jnp.where(mag_ref[...] < 0, corr_ref[0], 0.0)
    partial = jax.lax.dot_general(
        a, w_ref[...], (((1,), (0,)), ((), ())),
        precision=jax.lax.Precision.DEFAULT,
        preferred_element_type=jnp.float32)

    @pl.when(c == 0)
    def _():
        o_ref[0] = partial

    @pl.when(c != 0)
    def _():
        o_ref[0] = o_ref[0] + partial


def _topk_lin(corr, W1T):
    return pl.pallas_call(
        _topk_lin_kernel,
        grid=(B, D // CTOP),
        in_specs=[pl.BlockSpec((1, L, CTOP), lambda b, c: (b, 0, c)),
                  pl.BlockSpec((CTOP, D), lambda b, c: (c, 0))],
        out_specs=pl.BlockSpec((1, L, D), lambda b, c: (b, 0, 0)),
        out_shape=jax.ShapeDtypeStruct((B, L, D), jnp.float32),
        scratch_shapes=[pltpu.VMEM((L, CTOP), jnp.float32)],
    )(corr, W1T)


def _ln2_kernel(o_ref, s1_ref, trend_ref, b1_ref, g1_ref, be1_ref,
                gF_ref, beF_ref, out_ref):
    y = s1_ref[0] + (o_ref[0] + b1_ref[...])
    mu = jnp.mean(y, axis=-1, keepdims=True)
    var = jnp.mean((y - mu) ** 2, axis=-1, keepdims=True)
    s2 = (y - mu) / jnp.sqrt(var + EPS) * g1_ref[...] + be1_ref[...]
    z = s2 + trend_ref[0]
    mu2 = jnp.mean(z, axis=-1, keepdims=True)
    var2 = jnp.mean((z - mu2) ** 2, axis=-1, keepdims=True)
    out_ref[0] = (z - mu2) / jnp.sqrt(var2 + EPS) * gF_ref[...] + beF_ref[...]


def _ln2(o, s1, trend, b1, g1, be1, gF, beF):
    vec = lambda v: jnp.reshape(v, (1, D))
    return pl.pallas_call(
        _ln2_kernel,
        grid=(B, L // RB),
        in_specs=[pl.BlockSpec((1, RB, D), lambda b, r: (b, r, 0))] * 3
        + [pl.BlockSpec((1, D), lambda b, r: (0, 0))] * 5,
        out_specs=pl.BlockSpec((1, RB, D), lambda b, r: (b, r, 0)),
        out_shape=jax.ShapeDtypeStruct((B, L, D), jnp.float32),
    )(o, s1, trend, vec(b1), vec(g1), vec(be1), vec(gF), vec(beF))


def _decomp_kernel(x_ref, trend_ref, seas_ref, xp_ref):
    CB = x_ref.shape[2]
    xp_ref[PAD:PAD + L, :] = x_ref[0]
    for j in range(1, PAD + 1):
        xp_ref[PAD - j, :] = x_ref[0, j, :]
        xp_ref[PAD + L - 1 + j, :] = x_ref[0, L - 1 - j, :]
    RC = 256
    for r in range(0, L, RC):
        acc = xp_ref[r:r + RC, :]
        for i in range(1, KSIZE):
            acc = acc + xp_ref[r + i:r + i + RC, :]
        trend = acc / KSIZE
        trend_ref[0, r:r + RC, :] = trend
        seas_ref[0, r:r + RC, :] = x_ref[0, r:r + RC, :] - trend


def _decomp(x):
    CB = 512
    return pl.pallas_call(
        _decomp_kernel,
        grid=(B, D // CB),
        in_specs=[pl.BlockSpec((1, L, CB), lambda b, c: (b, 0, c))],
        out_specs=[pl.BlockSpec((1, L, CB), lambda b, c: (b, 0, c)),
                   pl.BlockSpec((1, L, CB), lambda b, c: (b, 0, c))],
        out_shape=[jax.ShapeDtypeStruct((B, L, D), jnp.float32),
                   jax.ShapeDtypeStruct((B, L, D), jnp.float32)],
        scratch_shapes=[pltpu.VMEM((L + 2 * PAD + 8, CB), jnp.float32)],
    )(x)


def _layer_norm(x, g, b):
    mu = jnp.mean(x, axis=-1, keepdims=True)
    var = jnp.mean((x - mu) ** 2, axis=-1, keepdims=True)
    return (x - mu) / jnp.sqrt(var + EPS) * g + b


def kernel(x, W0, b0, g0, be0, W1, b1, g1, be1, gF, beF):
    trend, seasonal = _decomp(x)

    bidx = jnp.arange(B)[:, None, None]
    cidx = jnp.arange(D)[None, :, None]
    Xf = jnp.fft.rfft(seasonal, axis=1)
    ACf = Xf * jnp.conj(Xf)
    corr = jnp.fft.irfft(ACf, n=L, axis=1)
    corr_t = jnp.transpose(corr, (0, 2, 1))
    idx = _topk_idx(corr)
    mask = jnp.zeros(corr_t.shape, corr_t.dtype).at[bidx, cidx, idx].set(1.0)
    a = jnp.transpose(corr_t * mask, (0, 2, 1))
    o = a @ W0.T + b0
    s1 = _layer_norm(seasonal + o, g0, be0)

    Xf2 = jnp.fft.rfft(s1, axis=1)
    ACf2 = Xf2 * jnp.conj(Xf2)
    corr2 = jnp.fft.irfft(ACf2, n=L, axis=1)
    o2 = _topk_lin(corr2, jnp.transpose(W1))
    return _ln2(o2, s1, trend, b1, g1, be1, gF, beF)
